# Initial kernel scaffold; baseline (speedup 1.0000x reference)
import jax
import jax.numpy as jnp
from jax.experimental import pallas as pl


def kernel(atom_node, force_node, dir_edge, dist_edge, edge_index, W_mn1, b_mn1, W_mn2, b_mn2, W_me, W_em1a, W_em1b, W_em2a, W_em2b, W_eu, ln_g, ln_b):
    raise NotImplementedError("write your pallas kernel here")


# speedup vs baseline: 9.5776x; 9.5776x over previous
import functools

import jax
import jax.numpy as jnp
from jax import lax
from jax.experimental import pallas as pl
from jax.experimental.pallas import tpu as pltpu
from jax.experimental.pallas import tpu_sc as plsc

N = 10000
E = 160000
F = 128
NB = 16
EP = 163840
Nb = EP // 128
NP1 = 10112
K = 128
KH = 64
Nb2 = EP // KH
KB = 32
Nb3 = EP // KB
H = 64
BP = 4096
SROW = 632


def _silu(x):
    return x * jax.nn.sigmoid(x)


def _pack_body(idx_ref, dir_ref, dist_ref, wme_ref, io_ref, i6_ref, i3_ref, da_ref, db_ref, mo_ref):
    i = pl.program_id(0)
    colmask = (lax.broadcasted_iota(jnp.int32, (1, BP), 1) + i * BP) < E
    idxm = jnp.where(colmask, idx_ref[...], 0)
    io_ref[...] = idxm.reshape(2, BP // K, K)
    i6_ref[...] = idxm.reshape(2, BP // KH, KH)
    i3_ref[...] = idxm.reshape(2, BP // KB, KB)
    rowmask = (lax.broadcasted_iota(jnp.int32, (BP, 1), 0) + i * BP) < E
    b = jnp.where(rowmask, dir_ref[...], 0.0)
    da_ref[...] = jnp.concatenate(
        [jnp.broadcast_to(b[:, 0:1], (BP, 16)),
         jnp.broadcast_to(b[:, 1:2], (BP, 16))], axis=1)
    db_ref[...] = jnp.broadcast_to(b[:, 2:3], (BP, 16))
    d = jnp.where(rowmask, dist_ref[...], 0.0)
    mo_ref[...] = jnp.dot(d.astype(jnp.bfloat16),
                          wme_ref[...].astype(jnp.bfloat16).T,
                          preferred_element_type=jnp.float32)


def _pack(edge_index, dir_edge, dist_edge, w_me):
    return pl.pallas_call(
        _pack_body,
        grid=(EP // BP,),
        in_specs=[
            pl.BlockSpec((2, BP), lambda i: (0, i)),
            pl.BlockSpec((BP, 3), lambda i: (i, 0)),
            pl.BlockSpec((BP, NB), lambda i: (i, 0)),
            pl.BlockSpec((F, NB), lambda i: (0, 0)),
        ],
        out_specs=[
            pl.BlockSpec((2, BP // K, K), lambda i: (0, i, 0)),
            pl.BlockSpec((2, BP // KH, KH), lambda i: (0, i, 0)),
            pl.BlockSpec((2, BP // KB, KB), lambda i: (0, i, 0)),
            pl.BlockSpec((BP, 32), lambda i: (i, 0)),
            pl.BlockSpec((BP, 16), lambda i: (i, 0)),
            pl.BlockSpec((BP, F), lambda i: (i, 0)),
        ],
        out_shape=[jax.ShapeDtypeStruct((2, Nb, K), jnp.int32),
                   jax.ShapeDtypeStruct((2, Nb2, KH), jnp.int32),
                   jax.ShapeDtypeStruct((2, Nb3, KB), jnp.int32),
                   jax.ShapeDtypeStruct((EP, 32), jnp.float32),
                   jax.ShapeDtypeStruct((EP, 16), jnp.float32),
                   jax.ShapeDtypeStruct((EP, F), jnp.float32)],
    )(edge_index, dir_edge, dist_edge, w_me)


def _node_mlp_body(a_ref, w1_ref, b1_ref, w2_ref, b2_ref, o_ref):
    x = a_ref[...]
    h = _silu(jnp.dot(x.astype(jnp.bfloat16),
                      w1_ref[...].astype(jnp.bfloat16).T,
                      preferred_element_type=jnp.float32) + b1_ref[...])
    o_ref[...] = (jnp.dot(h.astype(jnp.bfloat16),
                          w2_ref[...].astype(jnp.bfloat16).T,
                          preferred_element_type=jnp.float32) + b2_ref[...])


def _node_mlp(atom, w1, b1, w2, b2):
    B = 1000
    return pl.pallas_call(
        _node_mlp_body,
        grid=(N // B,),
        in_specs=[
            pl.BlockSpec((B, F), lambda i: (i, 0)),
            pl.BlockSpec((F, F), lambda i: (0, 0)),
            pl.BlockSpec((1, F), lambda i: (0, 0)),
            pl.BlockSpec((F, F), lambda i: (0, 0)),
            pl.BlockSpec((1, F), lambda i: (0, 0)),
        ],
        out_specs=pl.BlockSpec((B, F), lambda i: (i, 0)),
        out_shape=jax.ShapeDtypeStruct((N, F), jnp.float32),
    )(atom, w1, b1.reshape(1, F), w2, b2.reshape(1, F))


def _zero_vmem_rows(buf, width_groups):
    def zrow(i, _):
        for q in range(width_groups):
            buf[i, pl.ds(q * 16, 16)] = jnp.zeros((16,), jnp.float32)
        return 0
    lax.fori_loop(0, K, zrow, 0)


def _zero_acc(buf, acc, s):
    cb = buf.shape[0]
    nfull, tail = SROW // cb, SROW % cb

    def zacc(k, _):
        pltpu.sync_copy(buf, acc.at[pl.ds(s * SROW + k * cb, cb)])
        return 0
    lax.fori_loop(0, nfull, zacc, 0)
    if tail:
        pltpu.sync_copy(buf.at[pl.ds(0, tail)],
                        acc.at[pl.ds(s * SROW + nfull * cb, tail)])


def _copy_out_acc(acc, out_view, s, cb):
    nfull, tail = SROW // cb, SROW % cb

    def cout(k, _):
        lo = s * SROW + k * cb
        pltpu.sync_copy(acc.at[pl.ds(lo, cb)], out_view.at[pl.ds(lo, cb)])
        return 0
    lax.fori_loop(0, nfull, cout, 0)
    if tail:
        lo = s * SROW + nfull * cb
        pltpu.sync_copy(acc.at[pl.ds(lo, tail)], out_view.at[pl.ds(lo, tail)])


def _sc_msg_body(idx2d, mnp_hbm, mep_hbm, msg_out, inv1_out,
                 ixs_a, ixd_a, ms_a, md_a, mg_a,
                 ixs_b, ixd_b, ms_b, md_b, mg_b,
                 acc, s1a, s2a, sma, s1b, s2b, smb):
    c = lax.axis_index("c")
    s = lax.axis_index("s")
    wid = s * 2 + c

    _zero_vmem_rows(mg_a, 8)
    _zero_acc(mg_a, acc, s)
    plsc.subcore_barrier()

    r0 = wid * 160

    def issue(r, ixs, ixd, ms, md, mg, s1, s2, sm):
        pltpu.sync_copy(idx2d.at[0, pl.ds(r, 1)], ixs)
        pltpu.sync_copy(idx2d.at[1, pl.ds(r, 1)], ixd)
        pltpu.async_copy(mnp_hbm.at[ixs.at[0]], ms, s1)
        pltpu.async_copy(mnp_hbm.at[ixd.at[0]], md, s2)
        pltpu.async_copy(mep_hbm.at[pl.ds(r * KB, KB)], mg, sm)

    def process(r, ixs, ixd, ms, md, mg, s1, s2, sm):
        pltpu.make_async_copy(mnp_hbm.at[ixs.at[0]], ms, s1).wait()
        pltpu.make_async_copy(mnp_hbm.at[ixd.at[0]], md, s2).wait()
        pltpu.make_async_copy(mep_hbm.at[pl.ds(r * KB, KB)], mg, sm).wait()

        @plsc.parallel_loop(0, KB, unroll=8)
        def row(i):
            for q in range(8):
                sl = pl.ds(q * 16, 16)
                mg[i, sl] = mg[i, sl] * ms[i, sl] * md[i, sl]
        pltpu.sync_copy(mg, msg_out.at[pl.ds(r * KB, KB)])
        pltpu.sync_copy(mg, acc.at[ixs.at[0]], add=True)

    bufa = (ixs_a, ixd_a, ms_a, md_a, mg_a, s1a, s2a, sma)
    bufb = (ixs_b, ixd_b, ms_b, md_b, mg_b, s1b, s2b, smb)
    issue(r0, *bufa)
    issue(r0 + 1, *bufb)

    def pair(k, _):
        rr = r0 + 2 * k
        process(rr, *bufa)
        issue(rr + 2, *bufa)
        process(rr + 1, *bufb)
        issue(rr + 3, *bufb)
        return 0
    lax.fori_loop(0, 79, pair, 0)
    process(r0 + 158, *bufa)
    process(r0 + 159, *bufb)
    plsc.subcore_barrier()
    _copy_out_acc(acc, inv1_out.at[c], s, K)


_sc_msg = functools.partial(
    pl.kernel,
    out_type=[jax.ShapeDtypeStruct((EP, F), jnp.float32),
              jax.ShapeDtypeStruct((2, NP1, F), jnp.float32)],
    mesh=plsc.VectorSubcoreMesh(core_axis_name="c", subcore_axis_name="s"),
    scratch_types=[
        pltpu.VMEM((1, KB), jnp.int32),
        pltpu.VMEM((1, KB), jnp.int32),
        pltpu.VMEM((KB, F), jnp.float32),
        pltpu.VMEM((KB, F), jnp.float32),
        pltpu.VMEM((KB, F), jnp.float32),
        pltpu.VMEM((1, KB), jnp.int32),
        pltpu.VMEM((1, KB), jnp.int32),
        pltpu.VMEM((KB, F), jnp.float32),
        pltpu.VMEM((KB, F), jnp.float32),
        pltpu.VMEM((KB, F), jnp.float32),
        pltpu.VMEM_SHARED((NP1, F), jnp.float32),
        pltpu.SemaphoreType.DMA,
        pltpu.SemaphoreType.DMA,
        pltpu.SemaphoreType.DMA,
        pltpu.SemaphoreType.DMA,
        pltpu.SemaphoreType.DMA,
        pltpu.SemaphoreType.DMA,
    ],
)(_sc_msg_body)


def _edge_mlp_body(m_ref, w1a, w1b, w2a, w2b, o_ref):
    m = m_ref[...].astype(jnp.bfloat16)
    h1 = _silu(jnp.dot(m, w1a[...].astype(jnp.bfloat16).T,
                       preferred_element_type=jnp.float32))
    e1 = jnp.dot(h1.astype(jnp.bfloat16), w1b[...].astype(jnp.bfloat16).T,
                 preferred_element_type=jnp.float32)
    h2 = _silu(jnp.dot(m, w2a[...].astype(jnp.bfloat16).T,
                       preferred_element_type=jnp.float32))
    e2 = jnp.dot(h2.astype(jnp.bfloat16), w2b[...].astype(jnp.bfloat16).T,
                 preferred_element_type=jnp.float32)
    o_ref[0] = jnp.concatenate([e1[:, :H], e2[:, :H]], axis=1)
    o_ref[1] = jnp.concatenate([e1[:, H:], e2[:, H:]], axis=1)


def _edge_mlp(msg, w1a, w1b, w2a, w2b):
    B = 512
    return pl.pallas_call(
        _edge_mlp_body,
        grid=(EP // B,),
        in_specs=[
            pl.BlockSpec((B, F), lambda i: (i, 0)),
            pl.BlockSpec((F, F), lambda i: (0, 0)),
            pl.BlockSpec((F, F), lambda i: (0, 0)),
            pl.BlockSpec((F, F), lambda i: (0, 0)),
            pl.BlockSpec((F, F), lambda i: (0, 0)),
        ],
        out_specs=pl.BlockSpec((2, B, F), lambda i: (0, i, 0)),
        out_shape=jax.ShapeDtypeStruct((2, EP, F), jnp.float32),
    )(msg, w1a, w1b, w2a, w2b)


def _ftab_body(f_ref, oa_ref, ob_ref):
    B = f_ref.shape[0]
    x = f_ref[...].reshape(B, 3, F)
    oa_ref[0] = jnp.concatenate([x[:, 0, :H], x[:, 1, :H]], axis=1)
    oa_ref[1] = jnp.concatenate([x[:, 0, H:], x[:, 1, H:]], axis=1)
    ob_ref[...] = x[:, 2, :]


def _ftab(f2d):
    B = 400
    return pl.pallas_call(
        _ftab_body,
        grid=(N // B,),
        in_specs=[pl.BlockSpec((B, 3 * F), lambda i: (i, 0))],
        out_specs=[
            pl.BlockSpec((2, B, F), lambda i: (0, i, 0)),
            pl.BlockSpec((B, F), lambda i: (i, 0)),
        ],
        out_shape=[jax.ShapeDtypeStruct((2, N, F), jnp.float32),
                   jax.ShapeDtypeStruct((N, F), jnp.float32)],
    )(f2d)


def _sc_fa_body(idx2d, dirf, em12, fta, fupd_out,
                ixs_a, ixd_a, e12_a, fg_a, ixs_b, ixd_b, e12_b, fg_b,
                dirv, acc, sg_a, se_a, sg_b, se_b, semd):
    c = lax.axis_index("c")
    s = lax.axis_index("s")

    _zero_vmem_rows(fg_a, 8)
    _zero_acc(fg_a, acc, s)
    plsc.subcore_barrier()

    off = c * N
    r0 = s * 160

    def issue(r, ixs, ixd, e12, fg, sg, se):
        pltpu.sync_copy(idx2d.at[0, pl.ds(r, 1)], ixs)
        pltpu.sync_copy(idx2d.at[1, pl.ds(r, 1)], ixd)
        for q in range(KH // 16):
            sl = pl.ds(q * 16, 16)
            ixd[0, sl] = ixd[0, sl] + off
        pltpu.async_copy(fta.at[ixd.at[0]], fg, sg)
        pltpu.async_copy(em12.at[c, pl.ds(r * KH, KH)], e12, se)

    def process(r, ixs, ixd, e12, fg, sg, se):
        pltpu.async_copy(dirf.at[pl.ds(r, 1)], dirv, semd)
        pltpu.make_async_copy(fta.at[ixd.at[0]], fg, sg).wait()
        pltpu.make_async_copy(em12.at[c, pl.ds(r * KH, KH)], e12, se).wait()
        pltpu.make_async_copy(dirf.at[pl.ds(r, 1)], dirv, semd).wait()

        @plsc.parallel_loop(0, KH, unroll=8)
        def row(i):
            d0 = dirv[0, pl.ds(i * 32, 16)]
            d1 = dirv[0, pl.ds(i * 32 + 16, 16)]
            for q in range(4):
                a = e12[i, pl.ds(q * 16, 16)]
                b = e12[i, pl.ds(H + q * 16, 16)]
                s0 = pl.ds(q * 16, 16)
                s1 = pl.ds(H + q * 16, 16)
                fg[i, s0] = a * d0 + b * fg[i, s0]
                fg[i, s1] = a * d1 + b * fg[i, s1]
        pltpu.sync_copy(fg, acc.at[ixs.at[0]], add=True)

    bufa = (ixs_a, ixd_a, e12_a, fg_a, sg_a, se_a)
    bufb = (ixs_b, ixd_b, e12_b, fg_b, sg_b, se_b)
    issue(r0, *bufa)
    issue(r0 + 1, *bufb)

    def pair(k, _):
        rr = r0 + 2 * k
        process(rr, *bufa)
        issue(rr + 2, *bufa)
        process(rr + 1, *bufb)
        issue(rr + 3, *bufb)
        return 0
    lax.fori_loop(0, 79, pair, 0)
    process(r0 + 158, *bufa)
    process(r0 + 159, *bufb)
    plsc.subcore_barrier()
    _copy_out_acc(acc, fupd_out.at[c], s, KH)


_sc_fa = functools.partial(
    pl.kernel,
    out_type=jax.ShapeDtypeStruct((2, NP1, F), jnp.float32),
    mesh=plsc.VectorSubcoreMesh(core_axis_name="c", subcore_axis_name="s"),
    scratch_types=[
        pltpu.VMEM((1, KH), jnp.int32),
        pltpu.VMEM((1, KH), jnp.int32),
        pltpu.VMEM((KH, F), jnp.float32),
        pltpu.VMEM((KH, F), jnp.float32),
        pltpu.VMEM((1, KH), jnp.int32),
        pltpu.VMEM((1, KH), jnp.int32),
        pltpu.VMEM((KH, F), jnp.float32),
        pltpu.VMEM((KH, F), jnp.float32),
        pltpu.VMEM((1, KH * 32), jnp.float32),
        pltpu.VMEM_SHARED((NP1, F), jnp.float32),
        pltpu.SemaphoreType.DMA,
        pltpu.SemaphoreType.DMA,
        pltpu.SemaphoreType.DMA,
        pltpu.SemaphoreType.DMA,
        pltpu.SemaphoreType.DMA,
    ],
)(_sc_fa_body)


def _sc_fb_body(idx2d, dirf, em12, ftb, dep, fupd_out,
                ixs_a, ixd_a, u_a, v_a, fg_a, ixs_b, ixd_b, u_b, v_b, fg_b,
                dirv, acc, sg_a, su_a, sv_a, sg_b, su_b, sv_b, semd):
    c = lax.axis_index("c")
    s = lax.axis_index("s")
    wid = s * 2 + c

    _zero_vmem_rows(fg_a, 8)
    _zero_acc(fg_a, acc, s)
    plsc.subcore_barrier()

    r0 = wid * 160

    def issue(r, ixs, ixd, u12, v12, fg, sg, su, sv):
        pltpu.sync_copy(idx2d.at[0, pl.ds(r, 1)], ixs)
        pltpu.sync_copy(idx2d.at[1, pl.ds(r, 1)], ixd)
        pltpu.async_copy(ftb.at[ixd.at[0]], fg, sg)
        pltpu.async_copy(em12.at[0, pl.ds(r * KB, KB)], u12, su)
        pltpu.async_copy(em12.at[1, pl.ds(r * KB, KB)], v12, sv)

    def process(r, ixs, ixd, u12, v12, fg, sg, su, sv):
        pltpu.async_copy(dirf.at[pl.ds(r, 1)], dirv, semd)
        pltpu.make_async_copy(ftb.at[ixd.at[0]], fg, sg).wait()
        pltpu.make_async_copy(em12.at[0, pl.ds(r * KB, KB)], u12, su).wait()
        pltpu.make_async_copy(em12.at[1, pl.ds(r * KB, KB)], v12, sv).wait()
        pltpu.make_async_copy(dirf.at[pl.ds(r, 1)], dirv, semd).wait()

        @plsc.parallel_loop(0, KB, unroll=8)
        def row(i):
            d2 = dirv[0, pl.ds(i * 16, 16)]
            for q in range(4):
                a = u12[i, pl.ds(q * 16, 16)]
                b = u12[i, pl.ds(H + q * 16, 16)]
                sl = pl.ds(q * 16, 16)
                fg[i, sl] = a * d2 + b * fg[i, sl]
                a2 = v12[i, pl.ds(q * 16, 16)]
                b2 = v12[i, pl.ds(H + q * 16, 16)]
                sh = pl.ds(H + q * 16, 16)
                fg[i, sh] = a2 * d2 + b2 * fg[i, sh]
        pltpu.sync_copy(fg, acc.at[ixs.at[0]], add=True)

    bufa = (ixs_a, ixd_a, u_a, v_a, fg_a, sg_a, su_a, sv_a)
    bufb = (ixs_b, ixd_b, u_b, v_b, fg_b, sg_b, su_b, sv_b)
    issue(r0, *bufa)
    issue(r0 + 1, *bufb)

    def pair(k, _):
        rr = r0 + 2 * k
        process(rr, *bufa)
        issue(rr + 2, *bufa)
        process(rr + 1, *bufb)
        issue(rr + 3, *bufb)
        return 0
    lax.fori_loop(0, 79, pair, 0)
    process(r0 + 158, *bufa)
    process(r0 + 159, *bufb)
    plsc.subcore_barrier()
    _copy_out_acc(acc, fupd_out.at[c], s, KH)


_sc_fb = functools.partial(
    pl.kernel,
    out_type=jax.ShapeDtypeStruct((2, NP1, F), jnp.float32),
    mesh=plsc.VectorSubcoreMesh(core_axis_name="c", subcore_axis_name="s"),
    scratch_types=[
        pltpu.VMEM((1, KB), jnp.int32),
        pltpu.VMEM((1, KB), jnp.int32),
        pltpu.VMEM((KB, F), jnp.float32),
        pltpu.VMEM((KB, F), jnp.float32),
        pltpu.VMEM((KB, F), jnp.float32),
        pltpu.VMEM((1, KB), jnp.int32),
        pltpu.VMEM((1, KB), jnp.int32),
        pltpu.VMEM((KB, F), jnp.float32),
        pltpu.VMEM((KB, F), jnp.float32),
        pltpu.VMEM((KB, F), jnp.float32),
        pltpu.VMEM((1, KB * 16), jnp.float32),
        pltpu.VMEM_SHARED((NP1, F), jnp.float32),
        pltpu.SemaphoreType.DMA,
        pltpu.SemaphoreType.DMA,
        pltpu.SemaphoreType.DMA,
        pltpu.SemaphoreType.DMA,
        pltpu.SemaphoreType.DMA,
        pltpu.SemaphoreType.DMA,
        pltpu.SemaphoreType.DMA,
    ],
)(_sc_fb_body)


def _final_body(a_ref, inv1_ref, f_ref, fa_ref, fb_ref, weu_ref, g_ref, b_ref,
                ao_ref, fo_ref):
    B = a_ref.shape[0]
    inv1 = inv1_ref[0] + inv1_ref[1]
    f = f_ref[...].reshape(B, 3, F)
    comp0 = jnp.concatenate([fa_ref[0][:, :H], fa_ref[1][:, :H]], axis=1)
    comp1 = jnp.concatenate([fa_ref[0][:, H:], fa_ref[1][:, H:]], axis=1)
    comp2 = fb_ref[0] + fb_ref[1]
    fu = jnp.stack([comp0, comp1, comp2], axis=1)
    fnew = f + fu
    fn2 = fnew.reshape(B * 3, F)
    t = jnp.dot(fn2.astype(jnp.bfloat16),
                weu_ref[...].astype(jnp.bfloat16).T,
                preferred_element_type=jnp.float32)
    inv2 = jnp.sum((fn2 * t).reshape(B, 3, F), axis=1)
    a = a_ref[...] + inv1 + inv2
    mu = jnp.mean(a, axis=-1, keepdims=True)
    var = jnp.mean((a - mu) ** 2, axis=-1, keepdims=True)
    ao_ref[...] = (a - mu) * jax.lax.rsqrt(var + 1e-5) * g_ref[...] + b_ref[...]
    fo_ref[...] = fnew.reshape(B, 3 * F)


def _finalize(atom, inv1, f2d, fupd_a, fupd_b, w_eu, ln_g, ln_b):
    B = 1000
    return pl.pallas_call(
        _final_body,
        grid=(N // B,),
        in_specs=[
            pl.BlockSpec((B, F), lambda i: (i, 0)),
            pl.BlockSpec((2, B, F), lambda i: (0, i, 0)),
            pl.BlockSpec((B, 3 * F), lambda i: (i, 0)),
            pl.BlockSpec((2, B, F), lambda i: (0, i, 0)),
            pl.BlockSpec((2, B, F), lambda i: (0, i, 0)),
            pl.BlockSpec((F, F), lambda i: (0, 0)),
            pl.BlockSpec((1, F), lambda i: (0, 0)),
            pl.BlockSpec((1, F), lambda i: (0, 0)),
        ],
        out_specs=[
            pl.BlockSpec((B, F), lambda i: (i, 0)),
            pl.BlockSpec((B, 3 * F), lambda i: (i, 0)),
        ],
        out_shape=[jax.ShapeDtypeStruct((N, F), jnp.float32),
                   jax.ShapeDtypeStruct((N, 3 * F), jnp.float32)],
    )(atom, inv1, f2d, fupd_a, fupd_b, w_eu, ln_g.reshape(1, F),
      ln_b.reshape(1, F))


def kernel(atom_node, force_node, dir_edge, dist_edge, edge_index,
           W_mn1, b_mn1, W_mn2, b_mn2, W_me, W_em1a, W_em1b, W_em2a, W_em2b,
           W_eu, ln_g, ln_b):
    idx2d, idx64, idx32, dira_e, dirb_e, mep = _pack(edge_index, dir_edge,
                                                     dist_edge, W_me)
    dira = dira_e.reshape(Nb2, KH * 32)
    dirb = dirb_e.reshape(Nb3, KB * 16)
    mnp = _node_mlp(atom_node, W_mn1, b_mn1, W_mn2, b_mn2)
    msg, inv1 = _sc_msg(idx32, mnp, mep)
    em12 = _edge_mlp(msg, W_em1a, W_em1b, W_em2a, W_em2b)

    f2d = force_node.reshape(N, 3 * F)
    ftab_a, ftab_b = _ftab(f2d)
    fta = ftab_a.reshape(2 * N, F)

    fupd_a = _sc_fa(idx64, dira, em12, fta)
    fupd_b = _sc_fb(idx32, dirb, em12, ftab_b, fupd_a)

    atom_out, force2d = _finalize(atom_node, inv1, f2d, fupd_a, fupd_b,
                                  W_eu, ln_g, ln_b)
    return atom_out, force2d.reshape(N, 3, F)


# speedup vs baseline: 9.9464x; 1.0385x over previous
import functools

import jax
import jax.numpy as jnp
from jax import lax
from jax.experimental import pallas as pl
from jax.experimental.pallas import tpu as pltpu
from jax.experimental.pallas import tpu_sc as plsc

N = 10000
E = 160000
F = 128
NB = 16
EP = 163840
Nb = EP // 128
NP1 = 10112
K = 128
KH = 64
Nb2 = EP // KH
KB = 32
Nb3 = EP // KB
H = 64
BP = 4096
SROW = 632


def _silu(x):
    return x * jax.nn.sigmoid(x)


def _pack_body(idx_ref, dir_ref, dist_ref, wme_ref, io_ref, i6_ref, i3_ref, da_ref, db_ref, mo_ref):
    i = pl.program_id(0)
    colmask = (lax.broadcasted_iota(jnp.int32, (1, BP), 1) + i * BP) < E
    idxm = jnp.where(colmask, idx_ref[...], 0)
    io_ref[...] = idxm.reshape(2, BP // K, K)
    i6_ref[...] = idxm.reshape(2, BP // KH, KH)
    i3_ref[...] = idxm.reshape(2, BP // KB, KB)
    rowmask = (lax.broadcasted_iota(jnp.int32, (BP, 1), 0) + i * BP) < E
    b = jnp.where(rowmask, dir_ref[...], 0.0)
    da_ref[...] = jnp.concatenate(
        [jnp.broadcast_to(b[:, 0:1], (BP, 16)),
         jnp.broadcast_to(b[:, 1:2], (BP, 16))], axis=1)
    db_ref[...] = jnp.broadcast_to(b[:, 2:3], (BP, 16))
    d = jnp.where(rowmask, dist_ref[...], 0.0)
    mo_ref[...] = jnp.dot(d.astype(jnp.bfloat16),
                          wme_ref[...].astype(jnp.bfloat16).T,
                          preferred_element_type=jnp.float32)


def _pack(edge_index, dir_edge, dist_edge, w_me):
    return pl.pallas_call(
        _pack_body,
        grid=(EP // BP,),
        in_specs=[
            pl.BlockSpec((2, BP), lambda i: (0, i)),
            pl.BlockSpec((BP, 3), lambda i: (i, 0)),
            pl.BlockSpec((BP, NB), lambda i: (i, 0)),
            pl.BlockSpec((F, NB), lambda i: (0, 0)),
        ],
        out_specs=[
            pl.BlockSpec((2, BP // K, K), lambda i: (0, i, 0)),
            pl.BlockSpec((2, BP // KH, KH), lambda i: (0, i, 0)),
            pl.BlockSpec((2, BP // KB, KB), lambda i: (0, i, 0)),
            pl.BlockSpec((BP, 32), lambda i: (i, 0)),
            pl.BlockSpec((BP, 16), lambda i: (i, 0)),
            pl.BlockSpec((BP, F), lambda i: (i, 0)),
        ],
        out_shape=[jax.ShapeDtypeStruct((2, Nb, K), jnp.int32),
                   jax.ShapeDtypeStruct((2, Nb2, KH), jnp.int32),
                   jax.ShapeDtypeStruct((2, Nb3, KB), jnp.int32),
                   jax.ShapeDtypeStruct((EP, 32), jnp.float32),
                   jax.ShapeDtypeStruct((EP, 16), jnp.float32),
                   jax.ShapeDtypeStruct((EP, F), jnp.float32)],
    )(edge_index, dir_edge, dist_edge, w_me)


def _node_mlp_body(a_ref, w1_ref, b1_ref, w2_ref, b2_ref, o_ref):
    x = a_ref[...]
    h = _silu(jnp.dot(x.astype(jnp.bfloat16),
                      w1_ref[...].astype(jnp.bfloat16).T,
                      preferred_element_type=jnp.float32) + b1_ref[...])
    o_ref[...] = (jnp.dot(h.astype(jnp.bfloat16),
                          w2_ref[...].astype(jnp.bfloat16).T,
                          preferred_element_type=jnp.float32) + b2_ref[...])


def _node_mlp(atom, w1, b1, w2, b2):
    B = 1000
    return pl.pallas_call(
        _node_mlp_body,
        grid=(N // B,),
        in_specs=[
            pl.BlockSpec((B, F), lambda i: (i, 0)),
            pl.BlockSpec((F, F), lambda i: (0, 0)),
            pl.BlockSpec((1, F), lambda i: (0, 0)),
            pl.BlockSpec((F, F), lambda i: (0, 0)),
            pl.BlockSpec((1, F), lambda i: (0, 0)),
        ],
        out_specs=pl.BlockSpec((B, F), lambda i: (i, 0)),
        out_shape=jax.ShapeDtypeStruct((N, F), jnp.float32),
    )(atom, w1, b1.reshape(1, F), w2, b2.reshape(1, F))


def _zero_vmem_rows(buf, width_groups):
    def zrow(i, _):
        for q in range(width_groups):
            buf[i, pl.ds(q * 16, 16)] = jnp.zeros((16,), jnp.float32)
        return 0
    lax.fori_loop(0, K, zrow, 0)


def _zero_acc(buf, acc, s):
    cb = buf.shape[0]
    nfull, tail = SROW // cb, SROW % cb

    def zacc(k, _):
        pltpu.sync_copy(buf, acc.at[pl.ds(s * SROW + k * cb, cb)])
        return 0
    lax.fori_loop(0, nfull, zacc, 0)
    if tail:
        pltpu.sync_copy(buf.at[pl.ds(0, tail)],
                        acc.at[pl.ds(s * SROW + nfull * cb, tail)])


def _copy_out_acc(acc, out_view, s, cb):
    nfull, tail = SROW // cb, SROW % cb

    def cout(k, _):
        lo = s * SROW + k * cb
        pltpu.sync_copy(acc.at[pl.ds(lo, cb)], out_view.at[pl.ds(lo, cb)])
        return 0
    lax.fori_loop(0, nfull, cout, 0)
    if tail:
        lo = s * SROW + nfull * cb
        pltpu.sync_copy(acc.at[pl.ds(lo, tail)], out_view.at[pl.ds(lo, tail)])


def _sc_msg_body(idx2d, mnp_hbm, mep_hbm, msg_out, inv1_out,
                 ixs_a, ixd_a, ms_a, md_a, mg_a,
                 ixs_b, ixd_b, ms_b, md_b, mg_b,
                 acc, s1a, s2a, sma, s1b, s2b, smb):
    c = lax.axis_index("c")
    s = lax.axis_index("s")
    wid = s * 2 + c

    _zero_vmem_rows(mg_a, 8)
    _zero_acc(mg_a, acc, s)
    plsc.subcore_barrier()

    r0 = wid * 160

    def issue(r, ixs, ixd, ms, md, mg, s1, s2, sm):
        pltpu.sync_copy(idx2d.at[0, pl.ds(r, 1)], ixs)
        pltpu.sync_copy(idx2d.at[1, pl.ds(r, 1)], ixd)
        pltpu.async_copy(mnp_hbm.at[ixs.at[0]], ms, s1)
        pltpu.async_copy(mnp_hbm.at[ixd.at[0]], md, s2)
        pltpu.async_copy(mep_hbm.at[pl.ds(r * KB, KB)], mg, sm)

    def process(r, ixs, ixd, ms, md, mg, s1, s2, sm):
        pltpu.make_async_copy(mnp_hbm.at[ixs.at[0]], ms, s1).wait()
        pltpu.make_async_copy(mnp_hbm.at[ixd.at[0]], md, s2).wait()
        pltpu.make_async_copy(mep_hbm.at[pl.ds(r * KB, KB)], mg, sm).wait()

        @plsc.parallel_loop(0, KB, unroll=4)
        def row(i):
            for q in range(8):
                sl = pl.ds(q * 16, 16)
                mg[i, sl] = mg[i, sl] * ms[i, sl] * md[i, sl]
        pltpu.sync_copy(mg, msg_out.at[pl.ds(r * KB, KB)])
        pltpu.sync_copy(mg, acc.at[ixs.at[0]], add=True)

    bufa = (ixs_a, ixd_a, ms_a, md_a, mg_a, s1a, s2a, sma)
    bufb = (ixs_b, ixd_b, ms_b, md_b, mg_b, s1b, s2b, smb)
    issue(r0, *bufa)
    issue(r0 + 1, *bufb)

    def pair(k, _):
        rr = r0 + 2 * k
        process(rr, *bufa)
        issue(rr + 2, *bufa)
        process(rr + 1, *bufb)
        issue(rr + 3, *bufb)
        return 0
    lax.fori_loop(0, 79, pair, 0)
    process(r0 + 158, *bufa)
    process(r0 + 159, *bufb)
    plsc.subcore_barrier()
    _copy_out_acc(acc, inv1_out.at[c], s, K)


_sc_msg = functools.partial(
    pl.kernel,
    out_type=[jax.ShapeDtypeStruct((EP, F), jnp.float32),
              jax.ShapeDtypeStruct((2, NP1, F), jnp.float32)],
    mesh=plsc.VectorSubcoreMesh(core_axis_name="c", subcore_axis_name="s"),
    scratch_types=[
        pltpu.VMEM((1, KB), jnp.int32),
        pltpu.VMEM((1, KB), jnp.int32),
        pltpu.VMEM((KB, F), jnp.float32),
        pltpu.VMEM((KB, F), jnp.float32),
        pltpu.VMEM((KB, F), jnp.float32),
        pltpu.VMEM((1, KB), jnp.int32),
        pltpu.VMEM((1, KB), jnp.int32),
        pltpu.VMEM((KB, F), jnp.float32),
        pltpu.VMEM((KB, F), jnp.float32),
        pltpu.VMEM((KB, F), jnp.float32),
        pltpu.VMEM_SHARED((NP1, F), jnp.float32),
        pltpu.SemaphoreType.DMA,
        pltpu.SemaphoreType.DMA,
        pltpu.SemaphoreType.DMA,
        pltpu.SemaphoreType.DMA,
        pltpu.SemaphoreType.DMA,
        pltpu.SemaphoreType.DMA,
    ],
)(_sc_msg_body)


def _edge_mlp_body(m_ref, w1a, w1b, w2a, w2b, o_ref):
    m = m_ref[...].astype(jnp.bfloat16)
    h1 = _silu(jnp.dot(m, w1a[...].astype(jnp.bfloat16).T,
                       preferred_element_type=jnp.float32))
    e1 = jnp.dot(h1.astype(jnp.bfloat16), w1b[...].astype(jnp.bfloat16).T,
                 preferred_element_type=jnp.float32)
    h2 = _silu(jnp.dot(m, w2a[...].astype(jnp.bfloat16).T,
                       preferred_element_type=jnp.float32))
    e2 = jnp.dot(h2.astype(jnp.bfloat16), w2b[...].astype(jnp.bfloat16).T,
                 preferred_element_type=jnp.float32)
    o_ref[0] = jnp.concatenate([e1[:, :H], e2[:, :H]], axis=1)
    o_ref[1] = jnp.concatenate([e1[:, H:], e2[:, H:]], axis=1)


def _edge_mlp(msg, w1a, w1b, w2a, w2b):
    B = 512
    return pl.pallas_call(
        _edge_mlp_body,
        grid=(EP // B,),
        in_specs=[
            pl.BlockSpec((B, F), lambda i: (i, 0)),
            pl.BlockSpec((F, F), lambda i: (0, 0)),
            pl.BlockSpec((F, F), lambda i: (0, 0)),
            pl.BlockSpec((F, F), lambda i: (0, 0)),
            pl.BlockSpec((F, F), lambda i: (0, 0)),
        ],
        out_specs=pl.BlockSpec((2, B, F), lambda i: (0, i, 0)),
        out_shape=jax.ShapeDtypeStruct((2, EP, F), jnp.float32),
    )(msg, w1a, w1b, w2a, w2b)


def _ftab_body(f_ref, oa_ref, ob_ref):
    B = f_ref.shape[0]
    x = f_ref[...].reshape(B, 3, F)
    oa_ref[0] = jnp.concatenate([x[:, 0, :H], x[:, 1, :H]], axis=1)
    oa_ref[1] = jnp.concatenate([x[:, 0, H:], x[:, 1, H:]], axis=1)
    ob_ref[...] = x[:, 2, :]


def _ftab(f2d):
    B = 400
    return pl.pallas_call(
        _ftab_body,
        grid=(N // B,),
        in_specs=[pl.BlockSpec((B, 3 * F), lambda i: (i, 0))],
        out_specs=[
            pl.BlockSpec((2, B, F), lambda i: (0, i, 0)),
            pl.BlockSpec((B, F), lambda i: (i, 0)),
        ],
        out_shape=[jax.ShapeDtypeStruct((2, N, F), jnp.float32),
                   jax.ShapeDtypeStruct((N, F), jnp.float32)],
    )(f2d)


def _sc_fa_body(idx2d, dirf, em12, fta, fupd_out,
                ixs_a, ixd_a, e12_a, fg_a, ixs_b, ixd_b, e12_b, fg_b,
                dirv, acc, sg_a, se_a, sg_b, se_b, semd):
    c = lax.axis_index("c")
    s = lax.axis_index("s")

    _zero_vmem_rows(fg_a, 8)
    _zero_acc(fg_a, acc, s)
    plsc.subcore_barrier()

    off = c * N
    r0 = s * 160

    def issue(r, ixs, ixd, e12, fg, sg, se):
        pltpu.sync_copy(idx2d.at[0, pl.ds(r, 1)], ixs)
        pltpu.sync_copy(idx2d.at[1, pl.ds(r, 1)], ixd)
        for q in range(KH // 16):
            sl = pl.ds(q * 16, 16)
            ixd[0, sl] = ixd[0, sl] + off
        pltpu.async_copy(fta.at[ixd.at[0]], fg, sg)
        pltpu.async_copy(em12.at[c, pl.ds(r * KH, KH)], e12, se)

    def process(r, ixs, ixd, e12, fg, sg, se):
        pltpu.async_copy(dirf.at[pl.ds(r, 1)], dirv, semd)
        pltpu.make_async_copy(fta.at[ixd.at[0]], fg, sg).wait()
        pltpu.make_async_copy(em12.at[c, pl.ds(r * KH, KH)], e12, se).wait()
        pltpu.make_async_copy(dirf.at[pl.ds(r, 1)], dirv, semd).wait()

        @plsc.parallel_loop(0, KH, unroll=4)
        def row(i):
            d0 = dirv[0, pl.ds(i * 32, 16)]
            d1 = dirv[0, pl.ds(i * 32 + 16, 16)]
            for q in range(4):
                a = e12[i, pl.ds(q * 16, 16)]
                b = e12[i, pl.ds(H + q * 16, 16)]
                s0 = pl.ds(q * 16, 16)
                s1 = pl.ds(H + q * 16, 16)
                fg[i, s0] = a * d0 + b * fg[i, s0]
                fg[i, s1] = a * d1 + b * fg[i, s1]
        pltpu.sync_copy(fg, acc.at[ixs.at[0]], add=True)

    bufa = (ixs_a, ixd_a, e12_a, fg_a, sg_a, se_a)
    bufb = (ixs_b, ixd_b, e12_b, fg_b, sg_b, se_b)
    issue(r0, *bufa)
    issue(r0 + 1, *bufb)

    def pair(k, _):
        rr = r0 + 2 * k
        process(rr, *bufa)
        issue(rr + 2, *bufa)
        process(rr + 1, *bufb)
        issue(rr + 3, *bufb)
        return 0
    lax.fori_loop(0, 79, pair, 0)
    process(r0 + 158, *bufa)
    process(r0 + 159, *bufb)
    plsc.subcore_barrier()
    _copy_out_acc(acc, fupd_out.at[c], s, KH)


_sc_fa = functools.partial(
    pl.kernel,
    out_type=jax.ShapeDtypeStruct((2, NP1, F), jnp.float32),
    mesh=plsc.VectorSubcoreMesh(core_axis_name="c", subcore_axis_name="s"),
    scratch_types=[
        pltpu.VMEM((1, KH), jnp.int32),
        pltpu.VMEM((1, KH), jnp.int32),
        pltpu.VMEM((KH, F), jnp.float32),
        pltpu.VMEM((KH, F), jnp.float32),
        pltpu.VMEM((1, KH), jnp.int32),
        pltpu.VMEM((1, KH), jnp.int32),
        pltpu.VMEM((KH, F), jnp.float32),
        pltpu.VMEM((KH, F), jnp.float32),
        pltpu.VMEM((1, KH * 32), jnp.float32),
        pltpu.VMEM_SHARED((NP1, F), jnp.float32),
        pltpu.SemaphoreType.DMA,
        pltpu.SemaphoreType.DMA,
        pltpu.SemaphoreType.DMA,
        pltpu.SemaphoreType.DMA,
        pltpu.SemaphoreType.DMA,
    ],
)(_sc_fa_body)


def _sc_fb_body(idx2d, dirf, em12, ftb, dep, fupd_out,
                ixs_a, ixd_a, u_a, v_a, fg_a, ixs_b, ixd_b, u_b, v_b, fg_b,
                dirv, acc, sg_a, su_a, sv_a, sg_b, su_b, sv_b, semd):
    c = lax.axis_index("c")
    s = lax.axis_index("s")
    wid = s * 2 + c

    _zero_vmem_rows(fg_a, 8)
    _zero_acc(fg_a, acc, s)
    plsc.subcore_barrier()

    r0 = wid * 160

    def issue(r, ixs, ixd, u12, v12, fg, sg, su, sv):
        pltpu.sync_copy(idx2d.at[0, pl.ds(r, 1)], ixs)
        pltpu.sync_copy(idx2d.at[1, pl.ds(r, 1)], ixd)
        pltpu.async_copy(ftb.at[ixd.at[0]], fg, sg)
        pltpu.async_copy(em12.at[0, pl.ds(r * KB, KB)], u12, su)
        pltpu.async_copy(em12.at[1, pl.ds(r * KB, KB)], v12, sv)

    def process(r, ixs, ixd, u12, v12, fg, sg, su, sv):
        pltpu.async_copy(dirf.at[pl.ds(r, 1)], dirv, semd)
        pltpu.make_async_copy(ftb.at[ixd.at[0]], fg, sg).wait()
        pltpu.make_async_copy(em12.at[0, pl.ds(r * KB, KB)], u12, su).wait()
        pltpu.make_async_copy(em12.at[1, pl.ds(r * KB, KB)], v12, sv).wait()
        pltpu.make_async_copy(dirf.at[pl.ds(r, 1)], dirv, semd).wait()

        @plsc.parallel_loop(0, KB, unroll=4)
        def row(i):
            d2 = dirv[0, pl.ds(i * 16, 16)]
            for q in range(4):
                a = u12[i, pl.ds(q * 16, 16)]
                b = u12[i, pl.ds(H + q * 16, 16)]
                sl = pl.ds(q * 16, 16)
                fg[i, sl] = a * d2 + b * fg[i, sl]
                a2 = v12[i, pl.ds(q * 16, 16)]
                b2 = v12[i, pl.ds(H + q * 16, 16)]
                sh = pl.ds(H + q * 16, 16)
                fg[i, sh] = a2 * d2 + b2 * fg[i, sh]
        pltpu.sync_copy(fg, acc.at[ixs.at[0]], add=True)

    bufa = (ixs_a, ixd_a, u_a, v_a, fg_a, sg_a, su_a, sv_a)
    bufb = (ixs_b, ixd_b, u_b, v_b, fg_b, sg_b, su_b, sv_b)
    issue(r0, *bufa)
    issue(r0 + 1, *bufb)

    def pair(k, _):
        rr = r0 + 2 * k
        process(rr, *bufa)
        issue(rr + 2, *bufa)
        process(rr + 1, *bufb)
        issue(rr + 3, *bufb)
        return 0
    lax.fori_loop(0, 79, pair, 0)
    process(r0 + 158, *bufa)
    process(r0 + 159, *bufb)
    plsc.subcore_barrier()
    _copy_out_acc(acc, fupd_out.at[c], s, KH)


_sc_fb = functools.partial(
    pl.kernel,
    out_type=jax.ShapeDtypeStruct((2, NP1, F), jnp.float32),
    mesh=plsc.VectorSubcoreMesh(core_axis_name="c", subcore_axis_name="s"),
    scratch_types=[
        pltpu.VMEM((1, KB), jnp.int32),
        pltpu.VMEM((1, KB), jnp.int32),
        pltpu.VMEM((KB, F), jnp.float32),
        pltpu.VMEM((KB, F), jnp.float32),
        pltpu.VMEM((KB, F), jnp.float32),
        pltpu.VMEM((1, KB), jnp.int32),
        pltpu.VMEM((1, KB), jnp.int32),
        pltpu.VMEM((KB, F), jnp.float32),
        pltpu.VMEM((KB, F), jnp.float32),
        pltpu.VMEM((KB, F), jnp.float32),
        pltpu.VMEM((1, KB * 16), jnp.float32),
        pltpu.VMEM_SHARED((NP1, F), jnp.float32),
        pltpu.SemaphoreType.DMA,
        pltpu.SemaphoreType.DMA,
        pltpu.SemaphoreType.DMA,
        pltpu.SemaphoreType.DMA,
        pltpu.SemaphoreType.DMA,
        pltpu.SemaphoreType.DMA,
        pltpu.SemaphoreType.DMA,
    ],
)(_sc_fb_body)


def _final_body(a_ref, inv1_ref, f_ref, fa_ref, fb_ref, weu_ref, g_ref, b_ref,
                ao_ref, fo_ref):
    B = a_ref.shape[0]
    inv1 = inv1_ref[0] + inv1_ref[1]
    f = f_ref[...].reshape(B, 3, F)
    comp0 = jnp.concatenate([fa_ref[0][:, :H], fa_ref[1][:, :H]], axis=1)
    comp1 = jnp.concatenate([fa_ref[0][:, H:], fa_ref[1][:, H:]], axis=1)
    comp2 = fb_ref[0] + fb_ref[1]
    fu = jnp.stack([comp0, comp1, comp2], axis=1)
    fnew = f + fu
    fn2 = fnew.reshape(B * 3, F)
    t = jnp.dot(fn2.astype(jnp.bfloat16),
                weu_ref[...].astype(jnp.bfloat16).T,
                preferred_element_type=jnp.float32)
    inv2 = jnp.sum((fn2 * t).reshape(B, 3, F), axis=1)
    a = a_ref[...] + inv1 + inv2
    mu = jnp.mean(a, axis=-1, keepdims=True)
    var = jnp.mean((a - mu) ** 2, axis=-1, keepdims=True)
    ao_ref[...] = (a - mu) * jax.lax.rsqrt(var + 1e-5) * g_ref[...] + b_ref[...]
    fo_ref[...] = fnew.reshape(B, 3 * F)


def _finalize(atom, inv1, f2d, fupd_a, fupd_b, w_eu, ln_g, ln_b):
    B = 1000
    return pl.pallas_call(
        _final_body,
        grid=(N // B,),
        in_specs=[
            pl.BlockSpec((B, F), lambda i: (i, 0)),
            pl.BlockSpec((2, B, F), lambda i: (0, i, 0)),
            pl.BlockSpec((B, 3 * F), lambda i: (i, 0)),
            pl.BlockSpec((2, B, F), lambda i: (0, i, 0)),
            pl.BlockSpec((2, B, F), lambda i: (0, i, 0)),
            pl.BlockSpec((F, F), lambda i: (0, 0)),
            pl.BlockSpec((1, F), lambda i: (0, 0)),
            pl.BlockSpec((1, F), lambda i: (0, 0)),
        ],
        out_specs=[
            pl.BlockSpec((B, F), lambda i: (i, 0)),
            pl.BlockSpec((B, 3 * F), lambda i: (i, 0)),
        ],
        out_shape=[jax.ShapeDtypeStruct((N, F), jnp.float32),
                   jax.ShapeDtypeStruct((N, 3 * F), jnp.float32)],
    )(atom, inv1, f2d, fupd_a, fupd_b, w_eu, ln_g.reshape(1, F),
      ln_b.reshape(1, F))


def kernel(atom_node, force_node, dir_edge, dist_edge, edge_index,
           W_mn1, b_mn1, W_mn2, b_mn2, W_me, W_em1a, W_em1b, W_em2a, W_em2b,
           W_eu, ln_g, ln_b):
    idx2d, idx64, idx32, dira_e, dirb_e, mep = _pack(edge_index, dir_edge,
                                                     dist_edge, W_me)
    dira = dira_e.reshape(Nb2, KH * 32)
    dirb = dirb_e.reshape(Nb3, KB * 16)
    mnp = _node_mlp(atom_node, W_mn1, b_mn1, W_mn2, b_mn2)
    msg, inv1 = _sc_msg(idx32, mnp, mep)
    em12 = _edge_mlp(msg, W_em1a, W_em1b, W_em2a, W_em2b)

    f2d = force_node.reshape(N, 3 * F)
    ftab_a, ftab_b = _ftab(f2d)
    fta = ftab_a.reshape(2 * N, F)

    fupd_a = _sc_fa(idx64, dira, em12, fta)
    fupd_b = _sc_fb(idx32, dirb, em12, ftab_b, fupd_a)

    atom_out, force2d = _finalize(atom_node, inv1, f2d, fupd_a, fupd_b,
                                  W_eu, ln_g, ln_b)
    return atom_out, force2d.reshape(N, 3, F)
